# Initial kernel scaffold; baseline (speedup 1.0000x reference)
import jax
import jax.numpy as jnp
from jax.experimental import pallas as pl


def kernel(x, edge_index, W1l, b1l, W1r, W2l, b2l, W2r, Wc, bc):
    raise NotImplementedError("write your pallas kernel here")


# speedup vs baseline: 44.7983x; 44.7983x over previous
import functools

import jax
import jax.numpy as jnp
from jax import lax
from jax.experimental import pallas as pl
from jax.experimental.pallas import tpu as pltpu
from jax.experimental.pallas import tpu_sc as plsc

N_NODES = 100000
N_EDGES = 6400000
NC = 2
NS = 16
N_WORKERS = NC * NS

LANE = 128
CHUNK_ROWS = 8
CHUNK_EDGES = CHUNK_ROWS * LANE
N_CHUNKS = -(-N_EDGES // (N_WORKERS * CHUNK_EDGES))
E_PAD = N_WORKERS * CHUNK_EDGES * N_CHUNKS
IDX_ROWS = E_PAD // LANE
ROWS_PER_W = IDX_ROWS // N_WORKERS

NPAD = 100352
DUMMY_SPREAD = 2048
ACC_ROWS = NPAD + DUMMY_SPREAD
FW = 16

SPAN = NPAD // NS
ZROWS = 784

_sc_mesh = plsc.VectorSubcoreMesh(core_axis_name="c", subcore_axis_name="s")


def _sc_agg_body(table, srci, dsti, out0, out1, srcv, dstv, rows, acc,
                 gsem, ssem):
    c = lax.axis_index("c")
    s = lax.axis_index("s")
    w = c * NS + s

    def zv_body(i, carry):
        rows[i, :] = jnp.zeros((16,), jnp.float32)
        return carry

    lax.fori_loop(0, CHUNK_EDGES, zv_body, 0)

    zbase = s * SPAN

    def zacc_body(i, carry):
        pltpu.sync_copy(rows.at[pl.ds(0, ZROWS)],
                        acc.at[pl.ds(zbase + i * ZROWS, ZROWS)])
        return carry

    lax.fori_loop(0, SPAN // ZROWS, zacc_body, 0)
    plsc.subcore_barrier()

    base = w * ROWS_PER_W

    def chunk_body(g, carry):
        r0 = base + g * CHUNK_ROWS
        pltpu.sync_copy(srci.at[pl.ds(r0, CHUNK_ROWS)], srcv)
        pltpu.sync_copy(dsti.at[pl.ds(r0, CHUNK_ROWS)], dstv)
        cps = [
            pltpu.async_copy(table.at[srcv.at[j]],
                             rows.at[pl.ds(j * LANE, LANE)], gsem)
            for j in range(CHUNK_ROWS)
        ]
        for cp in cps:
            cp.wait()
        cps = [
            pltpu.async_copy(rows.at[pl.ds(j * LANE, LANE)],
                             acc.at[dstv.at[j]], ssem, add=True)
            for j in range(CHUNK_ROWS)
        ]
        for cp in cps:
            cp.wait()
        return carry

    lax.fori_loop(0, N_CHUNKS, chunk_body, 0)
    plsc.subcore_barrier()

    @pl.when(c == 0)
    def _():
        pltpu.sync_copy(acc.at[pl.ds(zbase, SPAN)],
                        out0.at[pl.ds(zbase, SPAN)])

    @pl.when(c == 1)
    def _():
        pltpu.sync_copy(acc.at[pl.ds(zbase, SPAN)],
                        out1.at[pl.ds(zbase, SPAN)])


_sc_agg = functools.partial(
    pl.kernel,
    out_type=[
        jax.ShapeDtypeStruct((NPAD, FW), jnp.float32),
        jax.ShapeDtypeStruct((NPAD, FW), jnp.float32),
    ],
    mesh=_sc_mesh,
    scratch_types=[
        pltpu.VMEM((CHUNK_ROWS, LANE), jnp.int32),
        pltpu.VMEM((CHUNK_ROWS, LANE), jnp.int32),
        pltpu.VMEM((CHUNK_EDGES, FW), jnp.float32),
        pltpu.VMEM_SHARED((ACC_ROWS, FW), jnp.float32),
        pltpu.SemaphoreType.DMA,
        pltpu.SemaphoreType.DMA,
    ],
    compiler_params=pltpu.CompilerParams(use_tc_tiling_on_sc=False),
)(_sc_agg_body)


BLK = 1568
N_BLK = NPAD // BLK


def _stage1_body(p0, p1, xp, w1l, b1l, w1r, h1, deg):
    agg = p0[...] + p1[...]
    d = jnp.maximum(agg[:, 6:7], 1.0)
    x = xp[:, :6]
    pre = jnp.dot(agg[:, :6], w1l[...]) / d + b1l[...] + jnp.dot(x, w1r[...])
    h1[...] = jnp.maximum(pre, 0.0)
    deg[...] = d


def _stage2_body(q0, q1, h1, deg, w2l, b2l, w2r, wc, bc, out):
    agg = q0[...] + q1[...]
    d = deg[...]
    pre = jnp.dot(agg, w2l[...]) / d + b2l[...] + jnp.dot(h1[...], w2r[...])
    h2 = jnp.maximum(pre, 0.0)
    out[...] = jnp.dot(h2, wc[...]) + bc[...]


def _row_spec(width):
    return pl.BlockSpec((BLK, width), lambda i: (i, 0))


def _full_spec(shape):
    nd = len(shape)
    return pl.BlockSpec(shape, lambda i: (0,) * nd)


_stage1 = pl.pallas_call(
    _stage1_body,
    grid=(N_BLK,),
    in_specs=[
        _row_spec(FW), _row_spec(FW), _row_spec(FW),
        _full_spec((6, 16)), _full_spec((1, 16)), _full_spec((6, 16)),
    ],
    out_specs=[_row_spec(FW), _row_spec(1)],
    out_shape=[
        jax.ShapeDtypeStruct((NPAD, FW), jnp.float32),
        jax.ShapeDtypeStruct((NPAD, 1), jnp.float32),
    ],
)

_stage2 = pl.pallas_call(
    _stage2_body,
    grid=(N_BLK,),
    in_specs=[
        _row_spec(FW), _row_spec(FW), _row_spec(FW), _row_spec(1),
        _full_spec((16, 16)), _full_spec((1, 16)), _full_spec((16, 16)),
        _full_spec((16, 8)), _full_spec((1, 8)),
    ],
    out_specs=pl.BlockSpec((BLK, 8), lambda i: (i, 0)),
    out_shape=jax.ShapeDtypeStruct((NPAD, 8), jnp.float32),
)


def kernel(x, edge_index, W1l, b1l, W1r, W2l, b2l, W2r, Wc, bc):
    src = edge_index[0].astype(jnp.int32)
    dst = edge_index[1].astype(jnp.int32)
    pad_n = E_PAD - N_EDGES
    pad_i = jnp.arange(pad_n, dtype=jnp.int32)
    srci = jnp.concatenate([src, pad_i % N_NODES]).reshape(IDX_ROWS, LANE)
    dsti = jnp.concatenate([dst, NPAD + pad_i % DUMMY_SPREAD]).reshape(
        IDX_ROWS, LANE)

    xp = jnp.concatenate(
        [x, jnp.ones((N_NODES, 1), jnp.float32),
         jnp.zeros((N_NODES, FW - 7), jnp.float32)], axis=1)
    xp = jnp.pad(xp, ((0, NPAD - N_NODES), (0, 0)))

    p0, p1 = _sc_agg(xp, srci, dsti)
    h1, deg = _stage1(p0, p1, xp, W1l, b1l.reshape(1, 16), W1r)
    q0, q1 = _sc_agg(h1, srci, dsti)
    out = _stage2(q0, q1, h1, deg, W2l, b2l.reshape(1, 16), W2r, Wc,
                  bc.reshape(1, 8))
    return out[:N_NODES]


# speedup vs baseline: 72.2393x; 1.6125x over previous
import functools

import jax
import jax.numpy as jnp
from jax import lax
from jax.experimental import pallas as pl
from jax.experimental.pallas import tpu as pltpu
from jax.experimental.pallas import tpu_sc as plsc

N_NODES = 100000
N_EDGES = 6400000
NC = 2
NS = 16
N_WORKERS = NC * NS

LANE = 128
IDX_ROWS = N_EDGES // LANE
SUPER = 16
GROUPS = SUPER // 4
GROUP_EDGES = 4 * LANE
ROWS_PER_W = 1568
N_SUPER_FULL = ROWS_PER_W // SUPER
N_SUPER_LAST = (IDX_ROWS - 31 * ROWS_PER_W) // SUPER

NPAD = 100352
JUNK = 1024
ACC_ROWS = NPAD + JUNK

SPAN = NPAD // NS
ZREP = SPAN // GROUP_EDGES

_sc_mesh = plsc.VectorSubcoreMesh(core_axis_name="c", subcore_axis_name="s")


def _sc_agg_body(fw, stage_table, table, srci, dsti, zeros, out0, out1,
                 srcv, dstv, rows0, rows1, acc, tbl, gsem, ssem0, ssem1,
                 isem):
    c = lax.axis_index("c")
    s = lax.axis_index("s")
    w = c * NS + s
    rows = (rows0, rows1)
    ssem = (ssem0, ssem1)
    zbase = s * SPAN

    if stage_table:
        pltpu.sync_copy(table.at[pl.ds(zbase, SPAN)],
                        tbl.at[pl.ds(zbase, SPAN)])
        gsrc = tbl
    else:
        gsrc = table

    pltpu.sync_copy(zeros, rows0)

    def zacc_body(i, carry):
        pltpu.sync_copy(rows0, acc.at[pl.ds(zbase + i * GROUP_EDGES,
                                            GROUP_EDGES)])
        return carry

    lax.fori_loop(0, ZREP, zacc_body, 0)
    pltpu.sync_copy(rows0.at[pl.ds(0, SPAN - ZREP * GROUP_EDGES)],
                    acc.at[pl.ds(zbase + ZREP * GROUP_EDGES,
                                 SPAN - ZREP * GROUP_EDGES)])

    for slot in range(2):
        for k in range(4):
            pltpu.async_copy(
                rows[slot].at[pl.ds(k * LANE, LANE)],
                acc.at[pl.ds(NPAD + slot * GROUP_EDGES + k * LANE, LANE)],
                ssem[slot])

    base_row = w * ROWS_PER_W
    n_super = jnp.where(w == N_WORKERS - 1, N_SUPER_LAST, N_SUPER_FULL)

    pltpu.async_copy(srci.at[pl.ds(base_row * LANE, SUPER * LANE)],
                     srcv.at[pl.ds(0, SUPER * LANE)], isem)
    pltpu.async_copy(dsti.at[pl.ds(base_row, SUPER)],
                     dstv.at[pl.ds(0, SUPER)], isem)
    plsc.subcore_barrier()

    def super_body(i, carry):
        soff = (i & 1) * (SUPER * LANE)
        drow = (i & 1) * SUPER
        pltpu.make_async_copy(srci.at[pl.ds(0, SUPER * LANE)],
                             srcv.at[pl.ds(soff, SUPER * LANE)], isem).wait()
        pltpu.make_async_copy(dsti.at[pl.ds(0, SUPER)],
                             dstv.at[pl.ds(drow, SUPER)], isem).wait()

        @pl.when(i + 1 < n_super)
        def _():
            r1 = base_row + (i + 1) * SUPER
            soff1 = ((i + 1) & 1) * (SUPER * LANE)
            drow1 = ((i + 1) & 1) * SUPER
            pltpu.async_copy(srci.at[pl.ds(r1 * LANE, SUPER * LANE)],
                             srcv.at[pl.ds(soff1, SUPER * LANE)], isem)
            pltpu.async_copy(dsti.at[pl.ds(r1, SUPER)],
                             dstv.at[pl.ds(drow1, SUPER)], isem)

        for grp in range(GROUPS):
            slot = grp & 1
            for k in range(4):
                pltpu.make_async_copy(
                    table.at[pl.ds(0, LANE)],
                    rows[slot].at[pl.ds(k * LANE, LANE)],
                    ssem[slot]).wait()
            pltpu.async_copy(
                gsrc.at[srcv.at[pl.ds(soff + grp * GROUP_EDGES,
                                      GROUP_EDGES)]],
                rows[slot], gsem).wait()
            for k in range(4):
                pltpu.async_copy(rows[slot].at[pl.ds(k * LANE, LANE)],
                                 acc.at[dstv.at[drow + grp * 4 + k]],
                                 ssem[slot], add=True)
        return carry

    lax.fori_loop(0, n_super, super_body, 0)
    for slot in range(2):
        for k in range(4):
            pltpu.make_async_copy(table.at[pl.ds(0, LANE)],
                                 rows[slot].at[pl.ds(k * LANE, LANE)],
                                 ssem[slot]).wait()
    plsc.subcore_barrier()

    @pl.when(c == 0)
    def _():
        pltpu.sync_copy(acc.at[pl.ds(zbase, SPAN)],
                        out0.at[pl.ds(zbase, SPAN)])

    @pl.when(c == 1)
    def _():
        pltpu.sync_copy(acc.at[pl.ds(zbase, SPAN)],
                        out1.at[pl.ds(zbase, SPAN)])


def _make_sc_agg(fw, stage_table):
    scratch = [
        pltpu.VMEM((2 * SUPER * LANE,), jnp.int32),
        pltpu.VMEM((2 * SUPER, LANE), jnp.int32),
        pltpu.VMEM((GROUP_EDGES, fw), jnp.float32),
        pltpu.VMEM((GROUP_EDGES, fw), jnp.float32),
        pltpu.VMEM_SHARED((ACC_ROWS, fw), jnp.float32),
        pltpu.VMEM_SHARED((NPAD, fw) if stage_table else (8, fw),
                          jnp.float32),
        pltpu.SemaphoreType.DMA,
        pltpu.SemaphoreType.DMA,
        pltpu.SemaphoreType.DMA,
        pltpu.SemaphoreType.DMA,
    ]
    return functools.partial(
        pl.kernel,
        out_type=[
            jax.ShapeDtypeStruct((NPAD, fw), jnp.float32),
            jax.ShapeDtypeStruct((NPAD, fw), jnp.float32),
        ],
        mesh=_sc_mesh,
        scratch_types=scratch,
        compiler_params=pltpu.CompilerParams(use_tc_tiling_on_sc=False),
    )(functools.partial(_sc_agg_body, fw, stage_table))


_sc_agg8 = _make_sc_agg(8, stage_table=True)
_sc_agg16 = _make_sc_agg(16, stage_table=False)


BLK = 1568
N_BLK = NPAD // BLK


def _stage1_body(p0, p1, xp, w1l, b1l, w1r, h1, deg):
    agg = p0[...] + p1[...]
    d = jnp.maximum(agg[:, 6:7], 1.0)
    x = xp[:, :6]
    pre = jnp.dot(agg[:, :6], w1l[...]) / d + b1l[...] + jnp.dot(x, w1r[...])
    h1[...] = jnp.maximum(pre, 0.0)
    deg[...] = d


def _stage2_body(q0, q1, h1, deg, w2l, b2l, w2r, wc, bc, out):
    agg = q0[...] + q1[...]
    d = deg[...]
    pre = jnp.dot(agg, w2l[...]) / d + b2l[...] + jnp.dot(h1[...], w2r[...])
    h2 = jnp.maximum(pre, 0.0)
    out[...] = jnp.dot(h2, wc[...]) + bc[...]


def _row_spec(width):
    return pl.BlockSpec((BLK, width), lambda i: (i, 0))


def _full_spec(shape):
    nd = len(shape)
    return pl.BlockSpec(shape, lambda i: (0,) * nd)


_stage1 = pl.pallas_call(
    _stage1_body,
    grid=(N_BLK,),
    in_specs=[
        _row_spec(8), _row_spec(8), _row_spec(8),
        _full_spec((6, 16)), _full_spec((1, 16)), _full_spec((6, 16)),
    ],
    out_specs=[_row_spec(16), _row_spec(1)],
    out_shape=[
        jax.ShapeDtypeStruct((NPAD, 16), jnp.float32),
        jax.ShapeDtypeStruct((NPAD, 1), jnp.float32),
    ],
)

_stage2 = pl.pallas_call(
    _stage2_body,
    grid=(N_BLK,),
    in_specs=[
        _row_spec(16), _row_spec(16), _row_spec(16), _row_spec(1),
        _full_spec((16, 16)), _full_spec((1, 16)), _full_spec((16, 16)),
        _full_spec((16, 8)), _full_spec((1, 8)),
    ],
    out_specs=pl.BlockSpec((BLK, 8), lambda i: (i, 0)),
    out_shape=jax.ShapeDtypeStruct((NPAD, 8), jnp.float32),
)


def kernel(x, edge_index, W1l, b1l, W1r, W2l, b2l, W2r, Wc, bc):
    srci = edge_index[0].astype(jnp.int32)
    dsti = edge_index[1].astype(jnp.int32).reshape(IDX_ROWS, LANE)
    z8 = jnp.zeros((GROUP_EDGES, 8), jnp.float32)
    z16 = jnp.zeros((GROUP_EDGES, 16), jnp.float32)

    xp = jnp.concatenate(
        [x, jnp.ones((N_NODES, 1), jnp.float32),
         jnp.zeros((N_NODES, 1), jnp.float32)], axis=1)
    xp = jnp.pad(xp, ((0, NPAD - N_NODES), (0, 0)))

    p0, p1 = _sc_agg8(xp, srci, dsti, z8)
    h1, deg = _stage1(p0, p1, xp, W1l, b1l.reshape(1, 16), W1r)
    q0, q1 = _sc_agg16(h1, srci, dsti, z16)
    out = _stage2(q0, q1, h1, deg, W2l, b2l.reshape(1, 16), W2r, Wc,
                  bc.reshape(1, 8))
    return out[:N_NODES]


# speedup vs baseline: 113.4736x; 1.5708x over previous
import functools

import jax
import jax.numpy as jnp
from jax import lax
from jax.experimental import pallas as pl
from jax.experimental.pallas import tpu as pltpu
from jax.experimental.pallas import tpu_sc as plsc

N_NODES = 100000
N_EDGES = 6400000
NC = 2
NS = 16
N_WORKERS = NC * NS

LANE = 128
IDX_ROWS = N_EDGES // LANE
SUPER = 16
GROUPS = SUPER // 4
GROUP_EDGES = 4 * LANE
ROWS_PER_W = 1568
N_SUPER_FULL = ROWS_PER_W // SUPER
N_SUPER_LAST = (IDX_ROWS - 31 * ROWS_PER_W) // SUPER

NPAD = 100352
JUNK = 1024
ACC_ROWS = NPAD + JUNK

SPAN = NPAD // NS
ZREP = SPAN // GROUP_EDGES

_sc_mesh = plsc.VectorSubcoreMesh(core_axis_name="c", subcore_axis_name="s")


def _sc_agg_body(fw, stage_table, table, srci, dsti, zeros, out0, out1,
                 srcv, dstv, rows0, rows1, acc, tbl, gsem, ssem0, ssem1,
                 isem):
    c = lax.axis_index("c")
    s = lax.axis_index("s")
    w = c * NS + s
    rows = (rows0, rows1)
    ssem = (ssem0, ssem1)
    zbase = s * SPAN

    if stage_table:
        pltpu.sync_copy(table.at[pl.ds(zbase, SPAN)],
                        tbl.at[pl.ds(zbase, SPAN)])
        gsrc = tbl
    else:
        gsrc = table

    pltpu.sync_copy(zeros, rows0)

    def zacc_body(i, carry):
        pltpu.sync_copy(rows0, acc.at[pl.ds(zbase + i * GROUP_EDGES,
                                            GROUP_EDGES)])
        return carry

    lax.fori_loop(0, ZREP, zacc_body, 0)
    pltpu.sync_copy(rows0.at[pl.ds(0, SPAN - ZREP * GROUP_EDGES)],
                    acc.at[pl.ds(zbase + ZREP * GROUP_EDGES,
                                 SPAN - ZREP * GROUP_EDGES)])

    for slot in range(2):
        for k in range(4):
            pltpu.async_copy(
                rows[slot].at[pl.ds(k * LANE, LANE)],
                acc.at[pl.ds(NPAD + slot * GROUP_EDGES + k * LANE, LANE)],
                ssem[slot])

    base_row = w * ROWS_PER_W
    n_super = jnp.where(w == N_WORKERS - 1, N_SUPER_LAST, N_SUPER_FULL)

    pltpu.async_copy(srci.at[pl.ds(base_row * LANE, SUPER * LANE)],
                     srcv.at[pl.ds(0, SUPER * LANE)], isem)
    pltpu.async_copy(dsti.at[pl.ds(base_row, SUPER)],
                     dstv.at[pl.ds(0, SUPER)], isem)
    plsc.subcore_barrier()

    def super_body(i, carry):
        soff = (i & 1) * (SUPER * LANE)
        drow = (i & 1) * SUPER
        pltpu.make_async_copy(srci.at[pl.ds(0, SUPER * LANE)],
                             srcv.at[pl.ds(soff, SUPER * LANE)], isem).wait()
        pltpu.make_async_copy(dsti.at[pl.ds(0, SUPER)],
                             dstv.at[pl.ds(drow, SUPER)], isem).wait()

        @pl.when(i + 1 < n_super)
        def _():
            r1 = base_row + (i + 1) * SUPER
            soff1 = ((i + 1) & 1) * (SUPER * LANE)
            drow1 = ((i + 1) & 1) * SUPER
            pltpu.async_copy(srci.at[pl.ds(r1 * LANE, SUPER * LANE)],
                             srcv.at[pl.ds(soff1, SUPER * LANE)], isem)
            pltpu.async_copy(dsti.at[pl.ds(r1, SUPER)],
                             dstv.at[pl.ds(drow1, SUPER)], isem)

        for grp in range(GROUPS):
            slot = grp & 1
            for k in range(4):
                pltpu.make_async_copy(
                    table.at[pl.ds(0, LANE)],
                    rows[slot].at[pl.ds(k * LANE, LANE)],
                    ssem[slot]).wait()
            pltpu.async_copy(
                gsrc.at[srcv.at[pl.ds(soff + grp * GROUP_EDGES,
                                      GROUP_EDGES)]],
                rows[slot], gsem).wait()
            for k in range(4):
                pltpu.async_copy(rows[slot].at[pl.ds(k * LANE, LANE)],
                                 acc.at[dstv.at[drow + grp * 4 + k]],
                                 ssem[slot], add=True)
        return carry

    lax.fori_loop(0, n_super, super_body, 0)
    for slot in range(2):
        for k in range(4):
            pltpu.make_async_copy(table.at[pl.ds(0, LANE)],
                                 rows[slot].at[pl.ds(k * LANE, LANE)],
                                 ssem[slot]).wait()
    plsc.subcore_barrier()

    @pl.when(c == 0)
    def _():
        pltpu.sync_copy(acc.at[pl.ds(zbase, SPAN)],
                        out0.at[pl.ds(zbase, SPAN)])

    @pl.when(c == 1)
    def _():
        pltpu.sync_copy(acc.at[pl.ds(zbase, SPAN)],
                        out1.at[pl.ds(zbase, SPAN)])


def _make_sc_agg(fw, stage_table):
    scratch = [
        pltpu.VMEM((2 * SUPER * LANE,), jnp.int32),
        pltpu.VMEM((2 * SUPER, LANE), jnp.int32),
        pltpu.VMEM((GROUP_EDGES, fw), jnp.float32),
        pltpu.VMEM((GROUP_EDGES, fw), jnp.float32),
        pltpu.VMEM_SHARED((ACC_ROWS, fw), jnp.float32),
        pltpu.VMEM_SHARED((NPAD, fw) if stage_table else (8, fw),
                          jnp.float32),
        pltpu.SemaphoreType.DMA,
        pltpu.SemaphoreType.DMA,
        pltpu.SemaphoreType.DMA,
        pltpu.SemaphoreType.DMA,
    ]
    return functools.partial(
        pl.kernel,
        out_type=[
            jax.ShapeDtypeStruct((NPAD, fw), jnp.float32),
            jax.ShapeDtypeStruct((NPAD, fw), jnp.float32),
        ],
        mesh=_sc_mesh,
        scratch_types=scratch,
        compiler_params=pltpu.CompilerParams(use_tc_tiling_on_sc=False),
    )(functools.partial(_sc_agg_body, fw, stage_table))


_sc_agg8 = _make_sc_agg(8, stage_table=True)


PROWS = NPAD // 16
BLK = PROWS // 8
N_BLK = 8


def _stage1_body(p0, p1, xp, w1la, w1lb, w1ra, w1rb, s6, b1a, b1b,
                 h1a, h1b, recip):
    aggp = p0[...] + p1[...]
    degrep = jnp.dot(aggp, s6[...])
    r = 1.0 / jnp.maximum(degrep, 1.0)
    xpp = xp[...]
    h1a[...] = jnp.maximum(
        jnp.dot(aggp, w1la[...]) * r + b1a[...] + jnp.dot(xpp, w1ra[...]),
        0.0)
    h1b[...] = jnp.maximum(
        jnp.dot(aggp, w1lb[...]) * r + b1b[...] + jnp.dot(xpp, w1rb[...]),
        0.0)
    recip[...] = r


def _stage2_body(q0a, q1a, q0b, q1b, h1a, h1b, recip, w2la, w2lb, w2ra,
                 w2rb, rrep, wcbd, b2t, bct, out):
    qa = q0a[...] + q1a[...]
    qb = q0b[...] + q1b[...]
    r16 = jnp.dot(recip[...], rrep[...])
    h2 = jnp.maximum(
        (jnp.dot(qa, w2la[...]) + jnp.dot(qb, w2lb[...])) * r16 + b2t[...]
        + jnp.dot(h1a[...], w2ra[...]) + jnp.dot(h1b[...], w2rb[...]),
        0.0)
    out[...] = jnp.dot(h2, wcbd[...]) + bct[...]


def _row_spec(width=128):
    return pl.BlockSpec((BLK, width), lambda i: (i, 0))


def _full_spec(shape):
    nd = len(shape)
    return pl.BlockSpec(shape, lambda i: (0,) * nd)


_stage1 = pl.pallas_call(
    _stage1_body,
    grid=(N_BLK,),
    in_specs=[_row_spec(), _row_spec(), _row_spec()]
    + [_full_spec((128, 128))] * 5 + [_full_spec((1, 128))] * 2,
    out_specs=[_row_spec(), _row_spec(), _row_spec()],
    out_shape=[jax.ShapeDtypeStruct((PROWS, 128), jnp.float32)] * 3,
)

_stage2 = pl.pallas_call(
    _stage2_body,
    grid=(N_BLK,),
    in_specs=[_row_spec()] * 7
    + [_full_spec((128, 256))] * 5
    + [_full_spec((256, 128)), _full_spec((1, 256)), _full_spec((1, 128))],
    out_specs=_row_spec(),
    out_shape=jax.ShapeDtypeStruct((PROWS, 128), jnp.float32),
)


def _kron16(block):
    return jnp.kron(jnp.eye(16, dtype=jnp.float32), block)


def kernel(x, edge_index, W1l, b1l, W1r, W2l, b2l, W2r, Wc, bc):
    srci = edge_index[0].astype(jnp.int32)
    dsti = edge_index[1].astype(jnp.int32).reshape(IDX_ROWS, LANE)
    z8 = jnp.zeros((GROUP_EDGES, 8), jnp.float32)

    xp = jnp.concatenate(
        [x, jnp.ones((N_NODES, 1), jnp.float32),
         jnp.zeros((N_NODES, 1), jnp.float32)], axis=1)
    xp = jnp.pad(xp, ((0, NPAD - N_NODES), (0, 0)))
    xpp = xp.reshape(PROWS, 128)

    z2 = jnp.zeros((2, 8), jnp.float32)
    w1la = _kron16(jnp.concatenate([W1l[:, :8], z2]))
    w1lb = _kron16(jnp.concatenate([W1l[:, 8:], z2]))
    w1ra = _kron16(jnp.concatenate([W1r[:, :8], z2]))
    w1rb = _kron16(jnp.concatenate([W1r[:, 8:], z2]))
    s6 = _kron16(jnp.zeros((8, 8), jnp.float32).at[6, :].set(1.0))
    b1a = jnp.tile(b1l[:8], 16).reshape(1, 128)
    b1b = jnp.tile(b1l[8:], 16).reshape(1, 128)
    w2la = _kron16(W2l[:8, :])
    w2lb = _kron16(W2l[8:, :])
    w2ra = _kron16(W2r[:8, :])
    w2rb = _kron16(W2r[8:, :])
    rrep = _kron16(jnp.zeros((8, 16), jnp.float32).at[0, :].set(1.0))
    wcbd = _kron16(Wc)
    b2t = jnp.tile(b2l, 16).reshape(1, 256)
    bct = jnp.tile(bc, 16).reshape(1, 128)

    p0, p1 = _sc_agg8(xp, srci, dsti, z8)
    h1a, h1b, recip = _stage1(p0.reshape(PROWS, 128),
                              p1.reshape(PROWS, 128), xpp,
                              w1la, w1lb, w1ra, w1rb, s6, b1a, b1b)
    q0a, q1a = _sc_agg8(h1a.reshape(NPAD, 8), srci, dsti, z8)
    q0b, q1b = _sc_agg8(h1b.reshape(NPAD, 8), srci, dsti, z8)
    outp = _stage2(q0a.reshape(PROWS, 128), q1a.reshape(PROWS, 128),
                   q0b.reshape(PROWS, 128), q1b.reshape(PROWS, 128),
                   h1a, h1b, recip,
                   w2la, w2lb, w2ra, w2rb, rrep, wcbd, b2t, bct)
    return outp.reshape(NPAD, 8)[:N_NODES]


# speedup vs baseline: 117.8400x; 1.0385x over previous
import functools

import jax
import jax.numpy as jnp
from jax import lax
from jax.experimental import pallas as pl
from jax.experimental.pallas import tpu as pltpu
from jax.experimental.pallas import tpu_sc as plsc

N_NODES = 100000
N_EDGES = 6400000
NC = 2
NS = 16
N_WORKERS = NC * NS

LANE = 128
IDX_ROWS = N_EDGES // LANE
SUPER = 16
GROUP_ROWS = 8
GROUPS = SUPER // GROUP_ROWS
GROUP_EDGES = GROUP_ROWS * LANE
NSTR = GROUP_ROWS
ROWS_PER_W = 1568
N_SUPER_FULL = ROWS_PER_W // SUPER
N_SUPER_LAST = (IDX_ROWS - 31 * ROWS_PER_W) // SUPER

NPAD = 100352
JUNK = 2 * GROUP_EDGES
ACC_ROWS = NPAD + JUNK

SPAN = NPAD // NS
ZREP = SPAN // GROUP_EDGES

_sc_mesh = plsc.VectorSubcoreMesh(core_axis_name="c", subcore_axis_name="s")


def _sc_agg_body(fw, stage_table, table, srci, dsti, zeros, out0, out1,
                 srcv, dstv, rows0, rows1, acc, tbl, gsem, ssem0, ssem1,
                 isem):
    c = lax.axis_index("c")
    s = lax.axis_index("s")
    w = c * NS + s
    rows = (rows0, rows1)
    ssem = (ssem0, ssem1)
    zbase = s * SPAN

    if stage_table:
        pltpu.sync_copy(table.at[pl.ds(zbase, SPAN)],
                        tbl.at[pl.ds(zbase, SPAN)])
        gsrc = tbl
    else:
        gsrc = table

    pltpu.sync_copy(zeros, rows0)

    def zacc_body(i, carry):
        pltpu.sync_copy(rows0, acc.at[pl.ds(zbase + i * GROUP_EDGES,
                                            GROUP_EDGES)])
        return carry

    lax.fori_loop(0, ZREP, zacc_body, 0)
    pltpu.sync_copy(rows0.at[pl.ds(0, SPAN - ZREP * GROUP_EDGES)],
                    acc.at[pl.ds(zbase + ZREP * GROUP_EDGES,
                                 SPAN - ZREP * GROUP_EDGES)])

    for slot in range(2):
        for k in range(NSTR):
            pltpu.async_copy(
                rows[slot].at[pl.ds(k * LANE, LANE)],
                acc.at[pl.ds(NPAD + slot * GROUP_EDGES + k * LANE, LANE)],
                ssem[slot])

    base_row = w * ROWS_PER_W
    n_super = jnp.where(w == N_WORKERS - 1, N_SUPER_LAST, N_SUPER_FULL)

    pltpu.async_copy(srci.at[pl.ds(base_row * LANE, SUPER * LANE)],
                     srcv.at[pl.ds(0, SUPER * LANE)], isem)
    pltpu.async_copy(dsti.at[pl.ds(base_row, SUPER)],
                     dstv.at[pl.ds(0, SUPER)], isem)
    plsc.subcore_barrier()

    def super_body(i, carry):
        soff = (i & 1) * (SUPER * LANE)
        drow = (i & 1) * SUPER
        pltpu.make_async_copy(srci.at[pl.ds(0, SUPER * LANE)],
                             srcv.at[pl.ds(soff, SUPER * LANE)], isem).wait()
        pltpu.make_async_copy(dsti.at[pl.ds(0, SUPER)],
                             dstv.at[pl.ds(drow, SUPER)], isem).wait()

        @pl.when(i + 1 < n_super)
        def _():
            r1 = base_row + (i + 1) * SUPER
            soff1 = ((i + 1) & 1) * (SUPER * LANE)
            drow1 = ((i + 1) & 1) * SUPER
            pltpu.async_copy(srci.at[pl.ds(r1 * LANE, SUPER * LANE)],
                             srcv.at[pl.ds(soff1, SUPER * LANE)], isem)
            pltpu.async_copy(dsti.at[pl.ds(r1, SUPER)],
                             dstv.at[pl.ds(drow1, SUPER)], isem)

        for grp in range(GROUPS):
            slot = grp & 1
            for k in range(NSTR):
                pltpu.make_async_copy(
                    table.at[pl.ds(0, LANE)],
                    rows[slot].at[pl.ds(k * LANE, LANE)],
                    ssem[slot]).wait()
            pltpu.async_copy(
                gsrc.at[srcv.at[pl.ds(soff + grp * GROUP_EDGES,
                                      GROUP_EDGES)]],
                rows[slot], gsem).wait()
            for k in range(NSTR):
                pltpu.async_copy(rows[slot].at[pl.ds(k * LANE, LANE)],
                                 acc.at[dstv.at[drow + grp * NSTR + k]],
                                 ssem[slot], add=True)
        return carry

    lax.fori_loop(0, n_super, super_body, 0)
    for slot in range(2):
        for k in range(NSTR):
            pltpu.make_async_copy(table.at[pl.ds(0, LANE)],
                                 rows[slot].at[pl.ds(k * LANE, LANE)],
                                 ssem[slot]).wait()
    plsc.subcore_barrier()

    @pl.when(c == 0)
    def _():
        pltpu.sync_copy(acc.at[pl.ds(zbase, SPAN)],
                        out0.at[pl.ds(zbase, SPAN)])

    @pl.when(c == 1)
    def _():
        pltpu.sync_copy(acc.at[pl.ds(zbase, SPAN)],
                        out1.at[pl.ds(zbase, SPAN)])


def _make_sc_agg(fw, stage_table):
    scratch = [
        pltpu.VMEM((2 * SUPER * LANE,), jnp.int32),
        pltpu.VMEM((2 * SUPER, LANE), jnp.int32),
        pltpu.VMEM((GROUP_EDGES, fw), jnp.float32),
        pltpu.VMEM((GROUP_EDGES, fw), jnp.float32),
        pltpu.VMEM_SHARED((ACC_ROWS, fw), jnp.float32),
        pltpu.VMEM_SHARED((NPAD, fw) if stage_table else (8, fw),
                          jnp.float32),
        pltpu.SemaphoreType.DMA,
        pltpu.SemaphoreType.DMA,
        pltpu.SemaphoreType.DMA,
        pltpu.SemaphoreType.DMA,
    ]
    return functools.partial(
        pl.kernel,
        out_type=[
            jax.ShapeDtypeStruct((NPAD, fw), jnp.float32),
            jax.ShapeDtypeStruct((NPAD, fw), jnp.float32),
        ],
        mesh=_sc_mesh,
        scratch_types=scratch,
        compiler_params=pltpu.CompilerParams(use_tc_tiling_on_sc=False),
    )(functools.partial(_sc_agg_body, fw, stage_table))


_sc_agg8 = _make_sc_agg(8, stage_table=True)


PROWS = NPAD // 16
BLK = PROWS // 8
N_BLK = 8


def _stage1_body(p0, p1, xp, w1la, w1lb, w1ra, w1rb, s6, b1a, b1b,
                 h1a, h1b, recip):
    aggp = p0[...] + p1[...]
    degrep = jnp.dot(aggp, s6[...])
    r = 1.0 / jnp.maximum(degrep, 1.0)
    xpp = xp[...]
    h1a[...] = jnp.maximum(
        jnp.dot(aggp, w1la[...]) * r + b1a[...] + jnp.dot(xpp, w1ra[...]),
        0.0)
    h1b[...] = jnp.maximum(
        jnp.dot(aggp, w1lb[...]) * r + b1b[...] + jnp.dot(xpp, w1rb[...]),
        0.0)
    recip[...] = r


def _stage2_body(q0a, q1a, q0b, q1b, h1a, h1b, recip, w2la, w2lb, w2ra,
                 w2rb, rrep, wcbd, b2t, bct, out):
    qa = q0a[...] + q1a[...]
    qb = q0b[...] + q1b[...]
    r16 = jnp.dot(recip[...], rrep[...])
    h2 = jnp.maximum(
        (jnp.dot(qa, w2la[...]) + jnp.dot(qb, w2lb[...])) * r16 + b2t[...]
        + jnp.dot(h1a[...], w2ra[...]) + jnp.dot(h1b[...], w2rb[...]),
        0.0)
    out[...] = jnp.dot(h2, wcbd[...]) + bct[...]


def _row_spec(width=128):
    return pl.BlockSpec((BLK, width), lambda i: (i, 0))


def _full_spec(shape):
    nd = len(shape)
    return pl.BlockSpec(shape, lambda i: (0,) * nd)


_stage1 = pl.pallas_call(
    _stage1_body,
    grid=(N_BLK,),
    in_specs=[_row_spec(), _row_spec(), _row_spec()]
    + [_full_spec((128, 128))] * 5 + [_full_spec((1, 128))] * 2,
    out_specs=[_row_spec(), _row_spec(), _row_spec()],
    out_shape=[jax.ShapeDtypeStruct((PROWS, 128), jnp.float32)] * 3,
)

_stage2 = pl.pallas_call(
    _stage2_body,
    grid=(N_BLK,),
    in_specs=[_row_spec()] * 7
    + [_full_spec((128, 256))] * 5
    + [_full_spec((256, 128)), _full_spec((1, 256)), _full_spec((1, 128))],
    out_specs=_row_spec(),
    out_shape=jax.ShapeDtypeStruct((PROWS, 128), jnp.float32),
)


def _kron16(block):
    return jnp.kron(jnp.eye(16, dtype=jnp.float32), block)


def kernel(x, edge_index, W1l, b1l, W1r, W2l, b2l, W2r, Wc, bc):
    srci = edge_index[0].astype(jnp.int32)
    dsti = edge_index[1].astype(jnp.int32).reshape(IDX_ROWS, LANE)
    z8 = jnp.zeros((GROUP_EDGES, 8), jnp.float32)

    xp0 = jnp.concatenate(
        [x, jnp.ones((N_NODES, 1), jnp.float32),
         jnp.zeros((N_NODES, 1), jnp.float32)], axis=1)
    xpp = jnp.pad(xp0.reshape(N_NODES // 16, 128),
                  ((0, PROWS - N_NODES // 16), (0, 0)))
    xp = xpp.reshape(NPAD, 8)

    z2 = jnp.zeros((2, 8), jnp.float32)
    w1la = _kron16(jnp.concatenate([W1l[:, :8], z2]))
    w1lb = _kron16(jnp.concatenate([W1l[:, 8:], z2]))
    w1ra = _kron16(jnp.concatenate([W1r[:, :8], z2]))
    w1rb = _kron16(jnp.concatenate([W1r[:, 8:], z2]))
    s6 = _kron16(jnp.zeros((8, 8), jnp.float32).at[6, :].set(1.0))
    b1a = jnp.tile(b1l[:8], 16).reshape(1, 128)
    b1b = jnp.tile(b1l[8:], 16).reshape(1, 128)
    w2la = _kron16(W2l[:8, :])
    w2lb = _kron16(W2l[8:, :])
    w2ra = _kron16(W2r[:8, :])
    w2rb = _kron16(W2r[8:, :])
    rrep = _kron16(jnp.zeros((8, 16), jnp.float32).at[0, :].set(1.0))
    wcbd = _kron16(Wc)
    b2t = jnp.tile(b2l, 16).reshape(1, 256)
    bct = jnp.tile(bc, 16).reshape(1, 128)

    p0, p1 = _sc_agg8(xp, srci, dsti, z8)
    h1a, h1b, recip = _stage1(p0.reshape(PROWS, 128),
                              p1.reshape(PROWS, 128), xpp,
                              w1la, w1lb, w1ra, w1rb, s6, b1a, b1b)
    q0a, q1a = _sc_agg8(h1a.reshape(NPAD, 8), srci, dsti, z8)
    q0b, q1b = _sc_agg8(h1b.reshape(NPAD, 8), srci, dsti, z8)
    outp = _stage2(q0a.reshape(PROWS, 128), q1a.reshape(PROWS, 128),
                   q0b.reshape(PROWS, 128), q1b.reshape(PROWS, 128),
                   h1a, h1b, recip,
                   w2la, w2lb, w2ra, w2rb, rrep, wcbd, b2t, bct)
    return outp.reshape(NPAD, 8)[:N_NODES]


# speedup vs baseline: 119.4129x; 1.0133x over previous
import functools

import jax
import jax.numpy as jnp
from jax import lax
from jax.experimental import pallas as pl
from jax.experimental.pallas import tpu as pltpu
from jax.experimental.pallas import tpu_sc as plsc

N_NODES = 100000
N_EDGES = 6400000
NC = 2
NS = 16
N_WORKERS = NC * NS

LANE = 128
IDX_ROWS = N_EDGES // LANE
SUPER = 16
GROUP_ROWS = 8
GROUPS = SUPER // GROUP_ROWS
GROUP_EDGES = GROUP_ROWS * LANE
NSTR = GROUP_ROWS
ROWS_PER_W = 1568
N_SUPER_FULL = ROWS_PER_W // SUPER
N_SUPER_LAST = (IDX_ROWS - 31 * ROWS_PER_W) // SUPER

NPAD = 100352
JUNK = 2 * GROUP_EDGES
ACC_ROWS = NPAD + JUNK

SPAN = NPAD // NS
ZREP = SPAN // GROUP_EDGES

_sc_mesh = plsc.VectorSubcoreMesh(core_axis_name="c", subcore_axis_name="s")


def _sc_agg_body(fw, stage_table, table, srci, dsti, zeros, out0, out1,
                 srcv, dstv, rows0, rows1, acc, tbl, gsem, ssem0, ssem1,
                 isem):
    c = lax.axis_index("c")
    s = lax.axis_index("s")
    w = c * NS + s
    rows = (rows0, rows1)
    ssem = (ssem0, ssem1)
    zbase = s * SPAN

    if stage_table:
        pltpu.sync_copy(table.at[pl.ds(zbase, SPAN)],
                        tbl.at[pl.ds(zbase, SPAN)])
        gsrc = tbl
    else:
        gsrc = table

    pltpu.sync_copy(zeros, rows0)

    def zacc_body(i, carry):
        pltpu.sync_copy(rows0, acc.at[pl.ds(zbase + i * GROUP_EDGES,
                                            GROUP_EDGES)])
        return carry

    lax.fori_loop(0, ZREP, zacc_body, 0)
    pltpu.sync_copy(rows0.at[pl.ds(0, SPAN - ZREP * GROUP_EDGES)],
                    acc.at[pl.ds(zbase + ZREP * GROUP_EDGES,
                                 SPAN - ZREP * GROUP_EDGES)])

    for slot in range(2):
        for k in range(NSTR):
            pltpu.async_copy(
                rows[slot].at[pl.ds(k * LANE, LANE)],
                acc.at[pl.ds(NPAD + slot * GROUP_EDGES + k * LANE, LANE)],
                ssem[slot])

    base_row = w * ROWS_PER_W
    n_super = jnp.where(w == N_WORKERS - 1, N_SUPER_LAST, N_SUPER_FULL)

    pltpu.async_copy(srci.at[pl.ds(base_row * LANE, SUPER * LANE)],
                     srcv.at[pl.ds(0, SUPER * LANE)], isem)
    pltpu.async_copy(dsti.at[pl.ds(base_row, SUPER)],
                     dstv.at[pl.ds(0, SUPER)], isem)
    plsc.subcore_barrier()

    def super_body(i, carry):
        soff = (i & 1) * (SUPER * LANE)
        drow = (i & 1) * SUPER
        pltpu.make_async_copy(srci.at[pl.ds(0, SUPER * LANE)],
                             srcv.at[pl.ds(soff, SUPER * LANE)], isem).wait()
        pltpu.make_async_copy(dsti.at[pl.ds(0, SUPER)],
                             dstv.at[pl.ds(drow, SUPER)], isem).wait()

        @pl.when(i + 1 < n_super)
        def _():
            r1 = base_row + (i + 1) * SUPER
            soff1 = ((i + 1) & 1) * (SUPER * LANE)
            pltpu.async_copy(srci.at[pl.ds(r1 * LANE, SUPER * LANE)],
                             srcv.at[pl.ds(soff1, SUPER * LANE)], isem)

        for grp in range(GROUPS):
            slot = grp & 1
            for k in range(NSTR):
                pltpu.make_async_copy(
                    table.at[pl.ds(0, LANE)],
                    rows[slot].at[pl.ds(k * LANE, LANE)],
                    ssem[slot]).wait()
            if grp == GROUPS - 1:
                @pl.when(i + 1 < n_super)
                def _():
                    r1 = base_row + (i + 1) * SUPER
                    drow1 = ((i + 1) & 1) * SUPER
                    pltpu.async_copy(dsti.at[pl.ds(r1, SUPER)],
                                     dstv.at[pl.ds(drow1, SUPER)], isem)
            pltpu.async_copy(
                gsrc.at[srcv.at[pl.ds(soff + grp * GROUP_EDGES,
                                      GROUP_EDGES)]],
                rows[slot], gsem).wait()
            for k in range(NSTR):
                pltpu.async_copy(rows[slot].at[pl.ds(k * LANE, LANE)],
                                 acc.at[dstv.at[drow + grp * NSTR + k]],
                                 ssem[slot], add=True)
        return carry

    lax.fori_loop(0, n_super, super_body, 0)
    for slot in range(2):
        for k in range(NSTR):
            pltpu.make_async_copy(table.at[pl.ds(0, LANE)],
                                 rows[slot].at[pl.ds(k * LANE, LANE)],
                                 ssem[slot]).wait()
    plsc.subcore_barrier()

    @pl.when(c == 0)
    def _():
        pltpu.sync_copy(acc.at[pl.ds(zbase, SPAN)],
                        out0.at[pl.ds(zbase, SPAN)])

    @pl.when(c == 1)
    def _():
        pltpu.sync_copy(acc.at[pl.ds(zbase, SPAN)],
                        out1.at[pl.ds(zbase, SPAN)])


def _make_sc_agg(fw, stage_table):
    scratch = [
        pltpu.VMEM((2 * SUPER * LANE,), jnp.int32),
        pltpu.VMEM((2 * SUPER, LANE), jnp.int32),
        pltpu.VMEM((GROUP_EDGES, fw), jnp.float32),
        pltpu.VMEM((GROUP_EDGES, fw), jnp.float32),
        pltpu.VMEM_SHARED((ACC_ROWS, fw), jnp.float32),
        pltpu.VMEM_SHARED((NPAD, fw) if stage_table else (8, fw),
                          jnp.float32),
        pltpu.SemaphoreType.DMA,
        pltpu.SemaphoreType.DMA,
        pltpu.SemaphoreType.DMA,
        pltpu.SemaphoreType.DMA,
    ]
    return functools.partial(
        pl.kernel,
        out_type=[
            jax.ShapeDtypeStruct((NPAD, fw), jnp.float32),
            jax.ShapeDtypeStruct((NPAD, fw), jnp.float32),
        ],
        mesh=_sc_mesh,
        scratch_types=scratch,
        compiler_params=pltpu.CompilerParams(use_tc_tiling_on_sc=False),
    )(functools.partial(_sc_agg_body, fw, stage_table))


_sc_agg8 = _make_sc_agg(8, stage_table=True)


PROWS = NPAD // 16
BLK = PROWS // 8
N_BLK = 8


def _stage1_body(p0, p1, xp, w1la, w1lb, w1ra, w1rb, s6, b1a, b1b,
                 h1a, h1b, recip):
    aggp = p0[...] + p1[...]
    degrep = jnp.dot(aggp, s6[...])
    r = 1.0 / jnp.maximum(degrep, 1.0)
    xpp = xp[...]
    h1a[...] = jnp.maximum(
        jnp.dot(aggp, w1la[...]) * r + b1a[...] + jnp.dot(xpp, w1ra[...]),
        0.0)
    h1b[...] = jnp.maximum(
        jnp.dot(aggp, w1lb[...]) * r + b1b[...] + jnp.dot(xpp, w1rb[...]),
        0.0)
    recip[...] = r


def _stage2_body(q0a, q1a, q0b, q1b, h1a, h1b, recip, w2la, w2lb, w2ra,
                 w2rb, rrep, wcbd, b2t, bct, out):
    qa = q0a[...] + q1a[...]
    qb = q0b[...] + q1b[...]
    r16 = jnp.dot(recip[...], rrep[...])
    h2 = jnp.maximum(
        (jnp.dot(qa, w2la[...]) + jnp.dot(qb, w2lb[...])) * r16 + b2t[...]
        + jnp.dot(h1a[...], w2ra[...]) + jnp.dot(h1b[...], w2rb[...]),
        0.0)
    out[...] = jnp.dot(h2, wcbd[...]) + bct[...]


def _row_spec(width=128):
    return pl.BlockSpec((BLK, width), lambda i: (i, 0))


def _full_spec(shape):
    nd = len(shape)
    return pl.BlockSpec(shape, lambda i: (0,) * nd)


_stage1 = pl.pallas_call(
    _stage1_body,
    grid=(N_BLK,),
    in_specs=[_row_spec(), _row_spec(), _row_spec()]
    + [_full_spec((128, 128))] * 5 + [_full_spec((1, 128))] * 2,
    out_specs=[_row_spec(), _row_spec(), _row_spec()],
    out_shape=[jax.ShapeDtypeStruct((PROWS, 128), jnp.float32)] * 3,
)

_stage2 = pl.pallas_call(
    _stage2_body,
    grid=(N_BLK,),
    in_specs=[_row_spec()] * 7
    + [_full_spec((128, 256))] * 5
    + [_full_spec((256, 128)), _full_spec((1, 256)), _full_spec((1, 128))],
    out_specs=_row_spec(),
    out_shape=jax.ShapeDtypeStruct((PROWS, 128), jnp.float32),
)


def _kron16(block):
    return jnp.kron(jnp.eye(16, dtype=jnp.float32), block)


def kernel(x, edge_index, W1l, b1l, W1r, W2l, b2l, W2r, Wc, bc):
    srci = edge_index[0].astype(jnp.int32)
    dsti = edge_index[1].astype(jnp.int32).reshape(IDX_ROWS, LANE)
    z8 = jnp.zeros((GROUP_EDGES, 8), jnp.float32)

    xp0 = jnp.concatenate(
        [x, jnp.ones((N_NODES, 1), jnp.float32),
         jnp.zeros((N_NODES, 1), jnp.float32)], axis=1)
    xpp = jnp.pad(xp0.reshape(N_NODES // 16, 128),
                  ((0, PROWS - N_NODES // 16), (0, 0)))
    xp = xpp.reshape(NPAD, 8)

    z2 = jnp.zeros((2, 8), jnp.float32)
    w1la = _kron16(jnp.concatenate([W1l[:, :8], z2]))
    w1lb = _kron16(jnp.concatenate([W1l[:, 8:], z2]))
    w1ra = _kron16(jnp.concatenate([W1r[:, :8], z2]))
    w1rb = _kron16(jnp.concatenate([W1r[:, 8:], z2]))
    s6 = _kron16(jnp.zeros((8, 8), jnp.float32).at[6, :].set(1.0))
    b1a = jnp.tile(b1l[:8], 16).reshape(1, 128)
    b1b = jnp.tile(b1l[8:], 16).reshape(1, 128)
    w2la = _kron16(W2l[:8, :])
    w2lb = _kron16(W2l[8:, :])
    w2ra = _kron16(W2r[:8, :])
    w2rb = _kron16(W2r[8:, :])
    rrep = _kron16(jnp.zeros((8, 16), jnp.float32).at[0, :].set(1.0))
    wcbd = _kron16(Wc)
    b2t = jnp.tile(b2l, 16).reshape(1, 256)
    bct = jnp.tile(bc, 16).reshape(1, 128)

    p0, p1 = _sc_agg8(xp, srci, dsti, z8)
    h1a, h1b, recip = _stage1(p0.reshape(PROWS, 128),
                              p1.reshape(PROWS, 128), xpp,
                              w1la, w1lb, w1ra, w1rb, s6, b1a, b1b)
    q0a, q1a = _sc_agg8(h1a.reshape(NPAD, 8), srci, dsti, z8)
    q0b, q1b = _sc_agg8(h1b.reshape(NPAD, 8), srci, dsti, z8)
    outp = _stage2(q0a.reshape(PROWS, 128), q1a.reshape(PROWS, 128),
                   q0b.reshape(PROWS, 128), q1b.reshape(PROWS, 128),
                   h1a, h1b, recip,
                   w2la, w2lb, w2ra, w2rb, rrep, wcbd, b2t, bct)
    return outp.reshape(NPAD, 8)[:N_NODES]
